# baseline (speedup 1.0000x reference)
import functools

import jax
import jax.numpy as jnp
from jax import lax
from jax.experimental import pallas as pl
from jax.experimental.pallas import tpu as pltpu
from jax.experimental.pallas import tpu_sc as plsc

B, C, H, W = 32, 384, 48, 48
N = B * C * H * W
NC, NS = 2, 16
NW = NC * NS
CHUNK = N // NW

_mesh = plsc.VectorSubcoreMesh(core_axis_name="c", subcore_axis_name="s")


@functools.partial(
    pl.kernel,
    mesh=_mesh,
    out_type=jax.ShapeDtypeStruct((N,), jnp.float32),
)
def _copy_kernel(x_hbm, out_hbm):
    wid = lax.axis_index("s") * NC + lax.axis_index("c")
    base = pl.multiple_of(wid * CHUNK, 8)
    pltpu.sync_copy(x_hbm.at[pl.ds(base, CHUNK)],
                    out_hbm.at[pl.ds(base, CHUNK)])


def kernel(x):
    out = _copy_kernel(x.reshape(N))
    return out.reshape(B, C, H, W)


# speedup vs baseline: 5.0497x; 5.0497x over previous
import functools

import jax
import jax.numpy as jnp
from jax import lax
from jax.experimental import pallas as pl
from jax.experimental.pallas import tpu as pltpu
from jax.experimental.pallas import tpu_sc as plsc

B, C, H, W = 32, 384, 48, 48
N = B * C * H * W
NC, NS = 2, 16
NW = NC * NS
CHUNK = N // NW

BLK = 49152
NB = CHUNK // BLK

_mesh = plsc.VectorSubcoreMesh(core_axis_name="c", subcore_axis_name="s")


@functools.partial(
    pl.kernel,
    mesh=_mesh,
    out_type=jax.ShapeDtypeStruct((N,), jnp.float32),
    scratch_types=[
        pltpu.VMEM((BLK,), jnp.float32),
        pltpu.VMEM((BLK,), jnp.float32),
        pltpu.SemaphoreType.DMA,
        pltpu.SemaphoreType.DMA,
        pltpu.SemaphoreType.DMA,
        pltpu.SemaphoreType.DMA,
    ],
)
def _copy_kernel(x_hbm, out_hbm, buf0, buf1, si0, si1, so0, so1):
    wid = lax.axis_index("s") * NC + lax.axis_index("c")
    base = pl.multiple_of(wid * CHUNK, 8)
    bufs, sin, sout = [buf0, buf1], [si0, si1], [so0, so1]
    stores = [None, None]
    for i in range(NB):
        b = i % 2
        off = base + i * BLK
        if stores[b] is not None:
            stores[b].wait()
        ld = pltpu.make_async_copy(x_hbm.at[pl.ds(off, BLK)], bufs[b], sin[b])
        ld.start()
        ld.wait()
        stores[b] = pltpu.make_async_copy(bufs[b],
                                          out_hbm.at[pl.ds(off, BLK)], sout[b])
        stores[b].start()
    stores[0].wait()
    stores[1].wait()


def kernel(x):
    out = _copy_kernel(x.reshape(N))
    return out.reshape(B, C, H, W)
